# baseline (speedup 1.0000x reference)
import functools

import jax
import jax.numpy as jnp
from jax import lax
from jax.experimental import pallas as pl
from jax.experimental.pallas import tpu as pltpu
from jax.experimental.pallas import tpu_sc as plsc

NUM_CORES = 2
NUM_SUBCORES = 16
LANES = 16
NUM_WORKERS = NUM_CORES * NUM_SUBCORES

BATCH = 16384
EMBED_DIM = 64
ROWS_PER_WORKER = BATCH // NUM_WORKERS
CHUNKS = EMBED_DIM // LANES


def _dist_mult_body(head_hbm, rel_hbm, tail_hbm, ent_hbm, relemb_hbm,
                    out_hbm, hidx_v, ridx_v, tidx_v, h_rows, r_rows, t_rows,
                    part_v, out_v, sem):
    wid = lax.axis_index("s") * NUM_CORES + lax.axis_index("c")
    base = wid * ROWS_PER_WORKER

    pltpu.sync_copy(head_hbm.at[pl.ds(base, ROWS_PER_WORKER)], hidx_v)
    pltpu.sync_copy(rel_hbm.at[pl.ds(base, ROWS_PER_WORKER)], ridx_v)
    pltpu.sync_copy(tail_hbm.at[pl.ds(base, ROWS_PER_WORKER)], tidx_v)

    c1 = pltpu.async_copy(ent_hbm.at[hidx_v], h_rows, sem)
    c2 = pltpu.async_copy(relemb_hbm.at[ridx_v], r_rows, sem)
    c3 = pltpu.async_copy(ent_hbm.at[tidx_v], t_rows, sem)
    c1.wait()
    c2.wait()
    c3.wait()

    def row_body(b, _):
        acc = (h_rows[b, pl.ds(0, LANES)] * r_rows[b, pl.ds(0, LANES)]
               * t_rows[b, pl.ds(0, LANES)])
        for c in range(1, CHUNKS):
            acc = acc + (h_rows[b, pl.ds(c * LANES, LANES)]
                         * r_rows[b, pl.ds(c * LANES, LANES)]
                         * t_rows[b, pl.ds(c * LANES, LANES)])
        part_v[pl.ds(b * LANES, LANES)] = acc
        return _

    lax.fori_loop(0, ROWS_PER_WORKER, row_body, 0, unroll=4)

    lane_iota = lax.iota(jnp.int32, LANES)

    def red_body(g, _):
        flat_base = (g * LANES + lane_iota) * LANES
        acc = plsc.load_gather(part_v, [flat_base])
        for j in range(1, LANES):
            acc = acc + plsc.load_gather(part_v, [flat_base + j])
        out_v[pl.ds(g * LANES, LANES)] = acc
        return _

    lax.fori_loop(0, ROWS_PER_WORKER // LANES, red_body, 0, unroll=2)

    pltpu.sync_copy(out_v, out_hbm.at[pl.ds(base, ROWS_PER_WORKER)])


@functools.partial(jax.jit, static_argnames=())
def kernel(head, relation, tail, entity_emb, relation_emb):
    mesh = plsc.VectorSubcoreMesh(core_axis_name="c", subcore_axis_name="s")
    run = pl.kernel(
        _dist_mult_body,
        out_type=jax.ShapeDtypeStruct((BATCH,), jnp.float32),
        mesh=mesh,
        scratch_types=[
            pltpu.VMEM((ROWS_PER_WORKER,), jnp.int32),
            pltpu.VMEM((ROWS_PER_WORKER,), jnp.int32),
            pltpu.VMEM((ROWS_PER_WORKER,), jnp.int32),
            pltpu.VMEM((ROWS_PER_WORKER, EMBED_DIM), jnp.float32),
            pltpu.VMEM((ROWS_PER_WORKER, EMBED_DIM), jnp.float32),
            pltpu.VMEM((ROWS_PER_WORKER, EMBED_DIM), jnp.float32),
            pltpu.VMEM((ROWS_PER_WORKER * LANES,), jnp.float32),
            pltpu.VMEM((ROWS_PER_WORKER,), jnp.float32),
            pltpu.SemaphoreType.DMA,
        ],
        compiler_params=pltpu.CompilerParams(needs_layout_passes=False,
                                             use_tc_tiling_on_sc=False),
    )
    return run(head.astype(jnp.int32), relation.astype(jnp.int32),
               tail.astype(jnp.int32), entity_emb, relation_emb)
